# Initial kernel scaffold; baseline (speedup 1.0000x reference)
import functools

import jax
import jax.numpy as jnp
from jax import lax
from jax.experimental import pallas as pl
from jax.experimental.pallas import tpu as pltpu
from jax.experimental.pallas import tpu_sc as plsc

N = 10000
E = 320000
D = 128
B = 16

NPAD = 10240
EPAD = 327680
NTILES = 32
CHUNK = 128
EPT = EPAD // NTILES
NCHUNKS = EPT // CHUNK
RPT = NPAD // 16
RB = 1024
NBLK = NPAD // RB

_mesh = plsc.VectorSubcoreMesh(core_axis_name="c", subcore_axis_name="s")


def _deg_body(d_hbm, w_hbm, z_hbm, out_hbm, idx_v, wv_v, acc):
    cid = lax.axis_index("c")
    sid = lax.axis_index("s")
    wid = cid * 16 + sid
    pltpu.sync_copy(z_hbm.at[pl.ds(sid * RPT, RPT)], acc.at[pl.ds(sid * RPT, RPT)])
    plsc.subcore_barrier()
    base0 = wid * EPT

    @pl.loop(0, NCHUNKS)
    def _(ci):
        base = base0 + ci * CHUNK
        pltpu.sync_copy(d_hbm.at[pl.ds(base, CHUNK)], idx_v)
        pltpu.sync_copy(w_hbm.at[pl.ds(base, CHUNK)], wv_v)
        pltpu.sync_copy(wv_v, acc.at[idx_v], add=True)

    plsc.subcore_barrier()
    pltpu.sync_copy(acc.at[pl.ds(sid * RPT, RPT)],
                    out_hbm.at[cid, pl.ds(sid * RPT, RPT)])


def _deg_partials(d_idx, w, zeros1):
    k = pl.kernel(
        _deg_body,
        out_type=jax.ShapeDtypeStruct((2, NPAD), jnp.float32),
        mesh=_mesh,
        scratch_types=[
            pltpu.VMEM((CHUNK,), jnp.int32),
            pltpu.VMEM((CHUNK,), jnp.float32),
            pltpu.VMEM_SHARED((NPAD,), jnp.float32),
        ],
    )
    return k(d_idx, w, zeros1)


def _msg_body(s_hbm, d_hbm, w_hbm, hp_hbm, z_hbm, out_hbm,
              sidx_v, didx_v, wv_v, rows_v, acc):
    cid = lax.axis_index("c")
    sid = lax.axis_index("s")
    wid = cid * 16 + sid
    for zc in range(RPT // CHUNK):
        r0 = sid * RPT + zc * CHUNK
        pltpu.sync_copy(z_hbm.at[pl.ds(r0, CHUNK)], acc.at[pl.ds(r0, CHUNK)])
    plsc.subcore_barrier()
    base0 = wid * EPT

    @pl.loop(0, NCHUNKS)
    def _(ci):
        base = base0 + ci * CHUNK
        pltpu.sync_copy(s_hbm.at[pl.ds(base, CHUNK)], sidx_v)
        pltpu.sync_copy(w_hbm.at[pl.ds(base, CHUNK)], wv_v)
        pltpu.sync_copy(hp_hbm.at[sidx_v], rows_v)
        pltpu.sync_copy(d_hbm.at[pl.ds(base, CHUNK)], didx_v)

        @pl.loop(0, CHUNK, step=16)
        def _(c0):
            w16 = wv_v[pl.ds(c0, 16)]
            for l in range(16):
                wsp = jnp.take(w16, jnp.full((16,), l, jnp.int32), axis=0,
                               mode="promise_in_bounds")
                e = c0 + l
                for j in range(8):
                    rows_v[e, pl.ds(j * 16, 16)] = (
                        rows_v[e, pl.ds(j * 16, 16)] * wsp)

        pltpu.sync_copy(rows_v, acc.at[didx_v], add=True)

    plsc.subcore_barrier()
    for zc in range(RPT // CHUNK):
        r0 = sid * RPT + zc * CHUNK
        pltpu.sync_copy(acc.at[pl.ds(r0, CHUNK)],
                        out_hbm.at[cid, pl.ds(r0, CHUNK)])


def _msg_partials(s_idx, d_idx, w, hp, zeros2):
    k = pl.kernel(
        _msg_body,
        out_type=jax.ShapeDtypeStruct((2, NPAD, D), jnp.float32),
        mesh=_mesh,
        scratch_types=[
            pltpu.VMEM((CHUNK,), jnp.int32),
            pltpu.VMEM((CHUNK,), jnp.int32),
            pltpu.VMEM((CHUNK,), jnp.float32),
            pltpu.VMEM((CHUNK, D), jnp.float32),
            pltpu.VMEM_SHARED((NPAD, D), jnp.float32),
        ],
    )
    return k(s_idx, d_idx, w, hp, zeros2)


def _dis_body(degp_ref, out_ref):
    deg = degp_ref[0:1, :] + degp_ref[1:2, :] + 1.0
    dis = jnp.where(deg > 0, lax.rsqrt(jnp.maximum(deg, 1e-12)), 0.0)
    col = jnp.reshape(dis, (NPAD, 1))
    out_ref[...] = jnp.broadcast_to(col, (NPAD, D))


def _dis_bcast(degp):
    return pl.pallas_call(
        _dis_body,
        out_shape=jax.ShapeDtypeStruct((NPAD, D), jnp.float32),
    )(degp)


def _mm1_body(x_ref, w_ref, dis_ref, out_ref):
    h = jnp.dot(x_ref[...], w_ref[...], precision=lax.Precision.HIGHEST)
    out_ref[...] = h * dis_ref[...]


def _h1_prime(x_pad, W1, dis2):
    return pl.pallas_call(
        _mm1_body,
        grid=(NBLK,),
        in_specs=[
            pl.BlockSpec((RB, D), lambda i: (i, 0)),
            pl.BlockSpec((D, D), lambda i: (0, 0)),
            pl.BlockSpec((RB, D), lambda i: (i, 0)),
        ],
        out_specs=pl.BlockSpec((RB, D), lambda i: (i, 0)),
        out_shape=jax.ShapeDtypeStruct((NPAD, D), jnp.float32),
    )(x_pad, W1, dis2)


def _comb_body(sp_ref, hp_ref, dis_ref, b_ref, w_ref, out_ref):
    z = dis_ref[...] * (sp_ref[0] + sp_ref[1] + hp_ref[...]) + b_ref[...][None, :]
    a = jnp.maximum(z, 0.0)
    h = jnp.dot(a, w_ref[...], precision=lax.Precision.HIGHEST)
    out_ref[...] = h * dis_ref[...]


def _h2_prime(spart, hp1, dis2, b1, W2):
    return pl.pallas_call(
        _comb_body,
        grid=(NBLK,),
        in_specs=[
            pl.BlockSpec((2, RB, D), lambda i: (0, i, 0)),
            pl.BlockSpec((RB, D), lambda i: (i, 0)),
            pl.BlockSpec((RB, D), lambda i: (i, 0)),
            pl.BlockSpec((D,), lambda i: (0,)),
            pl.BlockSpec((D, D), lambda i: (0, 0)),
        ],
        out_specs=pl.BlockSpec((RB, D), lambda i: (i, 0)),
        out_shape=jax.ShapeDtypeStruct((NPAD, D), jnp.float32),
    )(spart, hp1, dis2, b1, W2)


def _final_body(sp_ref, hp_ref, dis_ref, b_ref, batch_ref,
                inter_ref, pooled_ref, sums_scr, cnt_scr):
    i = pl.program_id(0)
    z = dis_ref[...] * (sp_ref[0] + sp_ref[1] + hp_ref[...]) + b_ref[...][None, :]
    a = jnp.maximum(z, 0.0)
    inter_ref[...] = a
    classes = lax.broadcasted_iota(jnp.int32, (RB, D), 1)
    onehot = (batch_ref[...] == classes).astype(jnp.float32)
    psum = lax.dot_general(onehot, a, (((0,), (0,)), ((), ())),
                           precision=lax.Precision.HIGHEST)
    pcnt = lax.dot_general(onehot, jnp.ones((RB, D), jnp.float32),
                           (((0,), (0,)), ((), ())),
                           precision=lax.Precision.HIGHEST)

    @pl.when(i == 0)
    def _():
        sums_scr[...] = psum
        cnt_scr[...] = pcnt

    @pl.when(i > 0)
    def _():
        sums_scr[...] += psum
        cnt_scr[...] += pcnt

    @pl.when(i == NBLK - 1)
    def _():
        pooled_ref[...] = (sums_scr[...] / jnp.maximum(cnt_scr[...], 1.0))[:B, :]


def _final(spart, hp2, dis2, b2, batch2):
    return pl.pallas_call(
        _final_body,
        grid=(NBLK,),
        in_specs=[
            pl.BlockSpec((2, RB, D), lambda i: (0, i, 0)),
            pl.BlockSpec((RB, D), lambda i: (i, 0)),
            pl.BlockSpec((RB, D), lambda i: (i, 0)),
            pl.BlockSpec((D,), lambda i: (0,)),
            pl.BlockSpec((RB, D), lambda i: (i, 0)),
        ],
        out_specs=[
            pl.BlockSpec((RB, D), lambda i: (i, 0)),
            pl.BlockSpec((B, D), lambda i: (0, 0)),
        ],
        out_shape=[
            jax.ShapeDtypeStruct((NPAD, D), jnp.float32),
            jax.ShapeDtypeStruct((B, D), jnp.float32),
        ],
        scratch_shapes=[
            pltpu.VMEM((D, D), jnp.float32),
            pltpu.VMEM((D, D), jnp.float32),
        ],
    )(spart, hp2, dis2, b2, batch2)


def kernel(x, edge_index, edge_attr, batch, W1, b1, W2, b2):
    s_idx = jnp.concatenate([edge_index[0], jnp.zeros((EPAD - E,), jnp.int32)])
    d_idx = jnp.concatenate([edge_index[1], jnp.zeros((EPAD - E,), jnp.int32)])
    w = jnp.concatenate([edge_attr[:, 0], jnp.zeros((EPAD - E,), jnp.float32)])
    x_pad = jnp.concatenate([x, jnp.zeros((NPAD - N, D), jnp.float32)])
    batch_pad = jnp.concatenate(
        [batch, jnp.full((NPAD - N,), D - 1, jnp.int32)])
    batch2 = jnp.broadcast_to(batch_pad[:, None], (NPAD, D))
    zeros1 = jnp.zeros((NPAD,), jnp.float32)
    zeros2 = jnp.zeros((NPAD, D), jnp.float32)

    degp = _deg_partials(d_idx, w, zeros1)
    dis2 = _dis_bcast(degp)
    hp1 = _h1_prime(x_pad, W1, dis2)
    sp1 = _msg_partials(s_idx, d_idx, w, hp1, zeros2)
    hp2 = _h2_prime(sp1, hp1, dis2, b1, W2)
    sp2 = _msg_partials(s_idx, d_idx, w, hp2, zeros2)
    inter_pad, pooled = _final(sp2, hp2, dis2, b2, batch2)
    return inter_pad[:N], pooled


# speedup vs baseline: 6.3628x; 6.3628x over previous
import functools

import jax
import jax.numpy as jnp
from jax import lax
from jax.experimental import pallas as pl
from jax.experimental.pallas import tpu as pltpu
from jax.experimental.pallas import tpu_sc as plsc

N = 10000
E = 320000
D = 128
B = 16

NPAD = 10240
EPAD = 327680
NTILES = 32
CHUNK = 128
EPT = EPAD // NTILES
NCHUNKS = EPT // CHUNK
RPT = NPAD // 16
RB = 1024
NBLK = NPAD // RB

_mesh = plsc.VectorSubcoreMesh(core_axis_name="c", subcore_axis_name="s")


def _deg_body(d_hbm, w_hbm, z_hbm, out_hbm, idx_v, wv_v, acc):
    cid = lax.axis_index("c")
    sid = lax.axis_index("s")
    wid = cid * 16 + sid
    pltpu.sync_copy(z_hbm.at[pl.ds(sid * RPT, RPT)], acc.at[pl.ds(sid * RPT, RPT)])
    plsc.subcore_barrier()
    base0 = wid * EPT

    @pl.loop(0, NCHUNKS)
    def _(ci):
        base = base0 + ci * CHUNK
        pltpu.sync_copy(d_hbm.at[pl.ds(base, CHUNK)], idx_v)
        pltpu.sync_copy(w_hbm.at[pl.ds(base, CHUNK)], wv_v)
        pltpu.sync_copy(wv_v, acc.at[idx_v], add=True)

    plsc.subcore_barrier()
    pltpu.sync_copy(acc.at[pl.ds(sid * RPT, RPT)],
                    out_hbm.at[cid, pl.ds(sid * RPT, RPT)])


def _deg_partials(d_idx, w, zeros1):
    k = pl.kernel(
        _deg_body,
        out_type=jax.ShapeDtypeStruct((2, NPAD), jnp.float32),
        mesh=_mesh,
        scratch_types=[
            pltpu.VMEM((CHUNK,), jnp.int32),
            pltpu.VMEM((CHUNK,), jnp.float32),
            pltpu.VMEM_SHARED((NPAD,), jnp.float32),
        ],
    )
    return k(d_idx, w, zeros1)


def _msg_body(s_hbm, d_hbm, w_hbm, hp_hbm, z_hbm, out_hbm,
              sidx_v, didx_v, wv_v, rows_v, acc):
    cid = lax.axis_index("c")
    sid = lax.axis_index("s")
    wid = cid * 16 + sid
    for zc in range(RPT // CHUNK):
        r0 = sid * RPT + zc * CHUNK
        pltpu.sync_copy(z_hbm.at[pl.ds(r0, CHUNK)], acc.at[pl.ds(r0, CHUNK)])
    plsc.subcore_barrier()
    base0 = wid * EPT

    @pl.loop(0, NCHUNKS)
    def _(ci):
        base = base0 + ci * CHUNK
        pltpu.sync_copy(s_hbm.at[pl.ds(base, CHUNK)], sidx_v)
        pltpu.sync_copy(w_hbm.at[pl.ds(base, CHUNK)], wv_v)
        pltpu.sync_copy(hp_hbm.at[sidx_v], rows_v)
        pltpu.sync_copy(d_hbm.at[pl.ds(base, CHUNK)], didx_v)

        @pl.loop(0, CHUNK, step=16)
        def _(c0):
            w16 = wv_v[pl.ds(c0, 16)]
            for l in range(16):
                wsp = lax.gather(
                    w16, jnp.full((16, 1), l, jnp.int32),
                    dimension_numbers=lax.GatherDimensionNumbers(
                        offset_dims=(), collapsed_slice_dims=(0,),
                        start_index_map=(0,)),
                    slice_sizes=(1,),
                    mode=lax.GatherScatterMode.PROMISE_IN_BOUNDS)
                e = c0 + l
                for j in range(8):
                    rows_v[e, pl.ds(j * 16, 16)] = (
                        rows_v[e, pl.ds(j * 16, 16)] * wsp)

        pltpu.sync_copy(rows_v, acc.at[didx_v], add=True)

    plsc.subcore_barrier()
    for zc in range(RPT // CHUNK):
        r0 = sid * RPT + zc * CHUNK
        pltpu.sync_copy(acc.at[pl.ds(r0, CHUNK)],
                        out_hbm.at[cid, pl.ds(r0, CHUNK)])


def _msg_partials(s_idx, d_idx, w, hp, zeros2):
    k = pl.kernel(
        _msg_body,
        out_type=jax.ShapeDtypeStruct((2, NPAD, D), jnp.float32),
        mesh=_mesh,
        scratch_types=[
            pltpu.VMEM((CHUNK,), jnp.int32),
            pltpu.VMEM((CHUNK,), jnp.int32),
            pltpu.VMEM((CHUNK,), jnp.float32),
            pltpu.VMEM((CHUNK, D), jnp.float32),
            pltpu.VMEM_SHARED((NPAD, D), jnp.float32),
        ],
    )
    return k(s_idx, d_idx, w, hp, zeros2)


def _dis_body(degp_ref, out_ref):
    deg = degp_ref[0:1, :] + degp_ref[1:2, :] + 1.0
    dis = jnp.where(deg > 0, lax.rsqrt(jnp.maximum(deg, 1e-12)), 0.0)
    col = jnp.reshape(dis, (NPAD, 1))
    out_ref[...] = jnp.broadcast_to(col, (NPAD, D))


def _dis_bcast(degp):
    return pl.pallas_call(
        _dis_body,
        out_shape=jax.ShapeDtypeStruct((NPAD, D), jnp.float32),
    )(degp)


def _mm1_body(x_ref, w_ref, dis_ref, out_ref):
    h = jnp.dot(x_ref[...], w_ref[...], precision=lax.Precision.HIGHEST)
    out_ref[...] = h * dis_ref[...]


def _h1_prime(x_pad, W1, dis2):
    return pl.pallas_call(
        _mm1_body,
        grid=(NBLK,),
        in_specs=[
            pl.BlockSpec((RB, D), lambda i: (i, 0)),
            pl.BlockSpec((D, D), lambda i: (0, 0)),
            pl.BlockSpec((RB, D), lambda i: (i, 0)),
        ],
        out_specs=pl.BlockSpec((RB, D), lambda i: (i, 0)),
        out_shape=jax.ShapeDtypeStruct((NPAD, D), jnp.float32),
    )(x_pad, W1, dis2)


def _comb_body(sp_ref, hp_ref, dis_ref, b_ref, w_ref, out_ref):
    z = dis_ref[...] * (sp_ref[0] + sp_ref[1] + hp_ref[...]) + b_ref[...][None, :]
    a = jnp.maximum(z, 0.0)
    h = jnp.dot(a, w_ref[...], precision=lax.Precision.HIGHEST)
    out_ref[...] = h * dis_ref[...]


def _h2_prime(spart, hp1, dis2, b1, W2):
    return pl.pallas_call(
        _comb_body,
        grid=(NBLK,),
        in_specs=[
            pl.BlockSpec((2, RB, D), lambda i: (0, i, 0)),
            pl.BlockSpec((RB, D), lambda i: (i, 0)),
            pl.BlockSpec((RB, D), lambda i: (i, 0)),
            pl.BlockSpec((D,), lambda i: (0,)),
            pl.BlockSpec((D, D), lambda i: (0, 0)),
        ],
        out_specs=pl.BlockSpec((RB, D), lambda i: (i, 0)),
        out_shape=jax.ShapeDtypeStruct((NPAD, D), jnp.float32),
    )(spart, hp1, dis2, b1, W2)


def _final_body(sp_ref, hp_ref, dis_ref, b_ref, batch_ref,
                inter_ref, pooled_ref, sums_scr, cnt_scr):
    i = pl.program_id(0)
    z = dis_ref[...] * (sp_ref[0] + sp_ref[1] + hp_ref[...]) + b_ref[...][None, :]
    a = jnp.maximum(z, 0.0)
    inter_ref[...] = a
    classes = lax.broadcasted_iota(jnp.int32, (RB, D), 1)
    onehot = (batch_ref[...] == classes).astype(jnp.float32)
    psum = lax.dot_general(onehot, a, (((0,), (0,)), ((), ())),
                           precision=lax.Precision.HIGHEST)
    pcnt = lax.dot_general(onehot, jnp.ones((RB, D), jnp.float32),
                           (((0,), (0,)), ((), ())),
                           precision=lax.Precision.HIGHEST)

    @pl.when(i == 0)
    def _():
        sums_scr[...] = psum
        cnt_scr[...] = pcnt

    @pl.when(i > 0)
    def _():
        sums_scr[...] += psum
        cnt_scr[...] += pcnt

    @pl.when(i == NBLK - 1)
    def _():
        pooled_ref[...] = (sums_scr[...] / jnp.maximum(cnt_scr[...], 1.0))[:B, :]


def _final(spart, hp2, dis2, b2, batch2):
    return pl.pallas_call(
        _final_body,
        grid=(NBLK,),
        in_specs=[
            pl.BlockSpec((2, RB, D), lambda i: (0, i, 0)),
            pl.BlockSpec((RB, D), lambda i: (i, 0)),
            pl.BlockSpec((RB, D), lambda i: (i, 0)),
            pl.BlockSpec((D,), lambda i: (0,)),
            pl.BlockSpec((RB, D), lambda i: (i, 0)),
        ],
        out_specs=[
            pl.BlockSpec((RB, D), lambda i: (i, 0)),
            pl.BlockSpec((B, D), lambda i: (0, 0)),
        ],
        out_shape=[
            jax.ShapeDtypeStruct((NPAD, D), jnp.float32),
            jax.ShapeDtypeStruct((B, D), jnp.float32),
        ],
        scratch_shapes=[
            pltpu.VMEM((D, D), jnp.float32),
            pltpu.VMEM((D, D), jnp.float32),
        ],
    )(spart, hp2, dis2, b2, batch2)


def kernel(x, edge_index, edge_attr, batch, W1, b1, W2, b2):
    s_idx = jnp.concatenate([edge_index[0], jnp.zeros((EPAD - E,), jnp.int32)])
    d_idx = jnp.concatenate([edge_index[1], jnp.zeros((EPAD - E,), jnp.int32)])
    w = jnp.concatenate([edge_attr[:, 0], jnp.zeros((EPAD - E,), jnp.float32)])
    x_pad = jnp.concatenate([x, jnp.zeros((NPAD - N, D), jnp.float32)])
    batch_pad = jnp.concatenate(
        [batch, jnp.full((NPAD - N,), D - 1, jnp.int32)])
    batch2 = jnp.broadcast_to(batch_pad[:, None], (NPAD, D))
    zeros1 = jnp.zeros((NPAD,), jnp.float32)
    zeros2 = jnp.zeros((NPAD, D), jnp.float32)

    degp = _deg_partials(d_idx, w, zeros1)
    dis2 = _dis_bcast(degp)
    hp1 = _h1_prime(x_pad, W1, dis2)
    sp1 = _msg_partials(s_idx, d_idx, w, hp1, zeros2)
    hp2 = _h2_prime(sp1, hp1, dis2, b1, W2)
    sp2 = _msg_partials(s_idx, d_idx, w, hp2, zeros2)
    inter_pad, pooled = _final(sp2, hp2, dis2, b2, batch2)
    return inter_pad[:N], pooled


# speedup vs baseline: 8.1419x; 1.2796x over previous
import functools

import jax
import jax.numpy as jnp
from jax import lax
from jax.experimental import pallas as pl
from jax.experimental.pallas import tpu as pltpu
from jax.experimental.pallas import tpu_sc as plsc

N = 10000
E = 320000
D = 128
B = 16

NPAD = 10240
EPAD = 327680
NTILES = 32
CHUNK = 128
EPT = EPAD // NTILES
NCHUNKS = EPT // CHUNK
RPT = NPAD // 16
RB = 1024
NBLK = NPAD // RB

_mesh = plsc.VectorSubcoreMesh(core_axis_name="c", subcore_axis_name="s")


def _deg_body(d_hbm, w_hbm, z_hbm, out_hbm, idx_v, wv_v, acc, dsem):
    cid = lax.axis_index("c")
    sid = lax.axis_index("s")
    wid = cid * 16 + sid
    pltpu.sync_copy(z_hbm.at[pl.ds(sid * RPT, RPT)], acc.at[pl.ds(sid * RPT, RPT)])
    ch0 = wid * NCHUNKS
    pltpu.sync_copy(d_hbm.at[pl.ds(ch0, NCHUNKS)], idx_v)
    pltpu.sync_copy(w_hbm.at[pl.ds(ch0, NCHUNKS)], wv_v)
    plsc.subcore_barrier()

    @pl.loop(0, NCHUNKS, step=8)
    def _(ci):
        for k in range(8):
            pltpu.async_copy(wv_v.at[ci + k], acc.at[idx_v.at[ci + k]],
                             dsem, add=True)
        for k in range(8):
            pltpu.make_async_copy(wv_v.at[ci + k], acc.at[idx_v.at[ci + k]],
                                  dsem).wait()

    plsc.subcore_barrier()
    pltpu.sync_copy(acc.at[pl.ds(sid * RPT, RPT)],
                    out_hbm.at[cid, pl.ds(sid * RPT, RPT)])


def _deg_partials(d_idx2, w2, zeros1):
    k = pl.kernel(
        _deg_body,
        out_type=jax.ShapeDtypeStruct((2, NPAD), jnp.float32),
        mesh=_mesh,
        scratch_types=[
            pltpu.VMEM((NCHUNKS, CHUNK), jnp.int32),
            pltpu.VMEM((NCHUNKS, CHUNK), jnp.float32),
            pltpu.VMEM_SHARED((NPAD,), jnp.float32),
            pltpu.SemaphoreType.DMA,
        ],
    )
    return k(d_idx2, w2, zeros1)


def _msg_body(pk_hbm, w_hbm, hp_hbm, z_hbm, out_hbm,
              pk0, pk1, pk2, pk3, wv, rows0, rows1, acc,
              gsem0, gsem1, ssem0, ssem1, psem0, psem1, psem2, psem3):
    cid = lax.axis_index("c")
    sid = lax.axis_index("s")
    wid = cid * 16 + sid
    for zc in range(RPT // CHUNK):
        r0 = sid * RPT + zc * CHUNK
        pltpu.sync_copy(z_hbm.at[pl.ds(r0, CHUNK)], acc.at[pl.ds(r0, CHUNK)])
    plsc.subcore_barrier()

    ch0 = wid * NCHUNKS
    bufs = (rows0, rows1)
    gsems = (gsem0, gsem1)
    ssems = (ssem0, ssem1)
    pks = (pk0, pk1, pk2, pk3)
    psems = (psem0, psem1, psem2, psem3)

    pltpu.sync_copy(w_hbm.at[pl.ds(ch0, NCHUNKS)], wv)
    pltpu.sync_copy(pk_hbm.at[ch0], pk0)
    pltpu.async_copy(pk_hbm.at[ch0 + 1], pk1, psem1)
    pltpu.async_copy(hp_hbm.at[pk0.at[0]], rows0, gsem0)

    @pl.loop(0, NCHUNKS, step=4)
    def _(ci):
        for b in range(4):
            cur = ci + b
            A, B = bufs[b % 2], bufs[(b + 1) % 2]
            P = pks[b]
            pltpu.make_async_copy(hp_hbm.at[P.at[0]], A, gsems[b % 2]).wait()

            @pl.when(cur > 0)
            def _():
                Pm1 = pks[(b - 1) % 4]
                pltpu.make_async_copy(B, acc.at[Pm1.at[1]],
                                      ssems[(b + 1) % 2]).wait()

            @pl.when(cur + 1 < NCHUNKS)
            def _():
                Pp1 = pks[(b + 1) % 4]
                pltpu.make_async_copy(pk_hbm.at[ch0 + cur + 1], Pp1,
                                      psems[(b + 1) % 4]).wait()
                pltpu.async_copy(hp_hbm.at[Pp1.at[0]], B, gsems[(b + 1) % 2])

            @pl.when(cur + 2 < NCHUNKS)
            def _():
                Pp2 = pks[(b + 2) % 4]
                pltpu.async_copy(pk_hbm.at[ch0 + cur + 2], Pp2,
                                 psems[(b + 2) % 4])

            @pl.loop(0, CHUNK, step=16)
            def _(c0):
                w16 = wv[cur, pl.ds(c0, 16)]
                for l in range(16):
                    wsp = lax.gather(
                        w16, jnp.full((16, 1), l, jnp.int32),
                        dimension_numbers=lax.GatherDimensionNumbers(
                            offset_dims=(), collapsed_slice_dims=(0,),
                            start_index_map=(0,)),
                        slice_sizes=(1,),
                        mode=lax.GatherScatterMode.PROMISE_IN_BOUNDS)
                    e = c0 + l
                    for j in range(8):
                        A[e, pl.ds(j * 16, 16)] = (
                            A[e, pl.ds(j * 16, 16)] * wsp)

            pltpu.async_copy(A, acc.at[P.at[1]], ssems[b % 2], add=True)

    pltpu.make_async_copy(bufs[(NCHUNKS - 1) % 2],
                          acc.at[pks[(NCHUNKS - 1) % 4].at[1]],
                          ssems[(NCHUNKS - 1) % 2]).wait()

    plsc.subcore_barrier()
    for zc in range(RPT // CHUNK):
        r0 = sid * RPT + zc * CHUNK
        pltpu.sync_copy(acc.at[pl.ds(r0, CHUNK)],
                        out_hbm.at[cid, pl.ds(r0, CHUNK)])


def _msg_partials(pk, w2, hp, zeros2):
    k = pl.kernel(
        _msg_body,
        out_type=jax.ShapeDtypeStruct((2, NPAD, D), jnp.float32),
        mesh=_mesh,
        scratch_types=[
            pltpu.VMEM((2, CHUNK), jnp.int32),
            pltpu.VMEM((2, CHUNK), jnp.int32),
            pltpu.VMEM((2, CHUNK), jnp.int32),
            pltpu.VMEM((2, CHUNK), jnp.int32),
            pltpu.VMEM((NCHUNKS, CHUNK), jnp.float32),
            pltpu.VMEM((CHUNK, D), jnp.float32),
            pltpu.VMEM((CHUNK, D), jnp.float32),
            pltpu.VMEM_SHARED((NPAD, D), jnp.float32),
            pltpu.SemaphoreType.DMA,
            pltpu.SemaphoreType.DMA,
            pltpu.SemaphoreType.DMA,
            pltpu.SemaphoreType.DMA,
            pltpu.SemaphoreType.DMA,
            pltpu.SemaphoreType.DMA,
            pltpu.SemaphoreType.DMA,
            pltpu.SemaphoreType.DMA,
        ],
    )
    return k(pk, w2, hp, zeros2)


def _dis_body(degp_ref, out_ref):
    deg = degp_ref[0:1, :] + degp_ref[1:2, :] + 1.0
    dis = jnp.where(deg > 0, lax.rsqrt(jnp.maximum(deg, 1e-12)), 0.0)
    col = jnp.reshape(dis, (NPAD, 1))
    out_ref[...] = jnp.broadcast_to(col, (NPAD, D))


def _dis_bcast(degp):
    return pl.pallas_call(
        _dis_body,
        out_shape=jax.ShapeDtypeStruct((NPAD, D), jnp.float32),
    )(degp)


def _mm1_body(x_ref, w_ref, dis_ref, out_ref):
    h = jnp.dot(x_ref[...], w_ref[...], precision=lax.Precision.HIGHEST)
    out_ref[...] = h * dis_ref[...]


def _h1_prime(x_pad, W1, dis2):
    return pl.pallas_call(
        _mm1_body,
        grid=(NBLK,),
        in_specs=[
            pl.BlockSpec((RB, D), lambda i: (i, 0)),
            pl.BlockSpec((D, D), lambda i: (0, 0)),
            pl.BlockSpec((RB, D), lambda i: (i, 0)),
        ],
        out_specs=pl.BlockSpec((RB, D), lambda i: (i, 0)),
        out_shape=jax.ShapeDtypeStruct((NPAD, D), jnp.float32),
    )(x_pad, W1, dis2)


def _comb_body(sp_ref, hp_ref, dis_ref, b_ref, w_ref, out_ref):
    z = dis_ref[...] * (sp_ref[0] + sp_ref[1] + hp_ref[...]) + b_ref[...][None, :]
    a = jnp.maximum(z, 0.0)
    h = jnp.dot(a, w_ref[...], precision=lax.Precision.HIGHEST)
    out_ref[...] = h * dis_ref[...]


def _h2_prime(spart, hp1, dis2, b1, W2):
    return pl.pallas_call(
        _comb_body,
        grid=(NBLK,),
        in_specs=[
            pl.BlockSpec((2, RB, D), lambda i: (0, i, 0)),
            pl.BlockSpec((RB, D), lambda i: (i, 0)),
            pl.BlockSpec((RB, D), lambda i: (i, 0)),
            pl.BlockSpec((D,), lambda i: (0,)),
            pl.BlockSpec((D, D), lambda i: (0, 0)),
        ],
        out_specs=pl.BlockSpec((RB, D), lambda i: (i, 0)),
        out_shape=jax.ShapeDtypeStruct((NPAD, D), jnp.float32),
    )(spart, hp1, dis2, b1, W2)


def _final_body(sp_ref, hp_ref, dis_ref, b_ref, batch_ref,
                inter_ref, pooled_ref, sums_scr, cnt_scr):
    i = pl.program_id(0)
    z = dis_ref[...] * (sp_ref[0] + sp_ref[1] + hp_ref[...]) + b_ref[...][None, :]
    a = jnp.maximum(z, 0.0)
    inter_ref[...] = a
    classes = lax.broadcasted_iota(jnp.int32, (RB, D), 1)
    onehot = (batch_ref[...] == classes).astype(jnp.float32)
    psum = lax.dot_general(onehot, a, (((0,), (0,)), ((), ())),
                           precision=lax.Precision.HIGHEST)
    pcnt = lax.dot_general(onehot, jnp.ones((RB, D), jnp.float32),
                           (((0,), (0,)), ((), ())),
                           precision=lax.Precision.HIGHEST)

    @pl.when(i == 0)
    def _():
        sums_scr[...] = psum
        cnt_scr[...] = pcnt

    @pl.when(i > 0)
    def _():
        sums_scr[...] += psum
        cnt_scr[...] += pcnt

    @pl.when(i == NBLK - 1)
    def _():
        pooled_ref[...] = (sums_scr[...] / jnp.maximum(cnt_scr[...], 1.0))[:B, :]


def _final(spart, hp2, dis2, b2, batch2):
    return pl.pallas_call(
        _final_body,
        grid=(NBLK,),
        in_specs=[
            pl.BlockSpec((2, RB, D), lambda i: (0, i, 0)),
            pl.BlockSpec((RB, D), lambda i: (i, 0)),
            pl.BlockSpec((RB, D), lambda i: (i, 0)),
            pl.BlockSpec((D,), lambda i: (0,)),
            pl.BlockSpec((RB, D), lambda i: (i, 0)),
        ],
        out_specs=[
            pl.BlockSpec((RB, D), lambda i: (i, 0)),
            pl.BlockSpec((B, D), lambda i: (0, 0)),
        ],
        out_shape=[
            jax.ShapeDtypeStruct((NPAD, D), jnp.float32),
            jax.ShapeDtypeStruct((B, D), jnp.float32),
        ],
        scratch_shapes=[
            pltpu.VMEM((D, D), jnp.float32),
            pltpu.VMEM((D, D), jnp.float32),
        ],
    )(spart, hp2, dis2, b2, batch2)


def kernel(x, edge_index, edge_attr, batch, W1, b1, W2, b2):
    s_idx = jnp.concatenate(
        [edge_index[0], jnp.zeros((EPAD - E,), jnp.int32)]
    ).reshape(EPAD // CHUNK, CHUNK)
    d_idx = jnp.concatenate(
        [edge_index[1], jnp.zeros((EPAD - E,), jnp.int32)]
    ).reshape(EPAD // CHUNK, CHUNK)
    w = jnp.concatenate(
        [edge_attr[:, 0], jnp.zeros((EPAD - E,), jnp.float32)]
    ).reshape(EPAD // CHUNK, CHUNK)
    pk = jnp.stack([s_idx, d_idx], axis=1)
    x_pad = jnp.concatenate([x, jnp.zeros((NPAD - N, D), jnp.float32)])
    batch_pad = jnp.concatenate(
        [batch, jnp.full((NPAD - N,), D - 1, jnp.int32)])
    batch2 = jnp.broadcast_to(batch_pad[:, None], (NPAD, D))
    zeros1 = jnp.zeros((NPAD,), jnp.float32)
    zeros2 = jnp.zeros((NPAD, D), jnp.float32)

    degp = _deg_partials(d_idx, w, zeros1)
    dis2 = _dis_bcast(degp)
    hp1 = _h1_prime(x_pad, W1, dis2)
    sp1 = _msg_partials(pk, w, hp1, zeros2)
    hp2 = _h2_prime(sp1, hp1, dis2, b1, W2)
    sp2 = _msg_partials(pk, w, hp2, zeros2)
    inter_pad, pooled = _final(sp2, hp2, dis2, b2, batch2)
    return inter_pad[:N], pooled
